# baseline (speedup 1.0000x reference)
import functools

import jax
import jax.numpy as jnp
from jax import lax
from jax.experimental import pallas as pl
from jax.experimental.pallas import tpu as pltpu
from jax.experimental.pallas import tpu_sc as plsc

N, L, D = 1600, 77, 768
N_CTX = 8
CTX_POS = 1
NC, NS = 2, 16
NW = NC * NS
LANES = 16
CHD = D // LANES

SLABS = L - N_CTX
HALF = N // 2
QUART = N // 4
NCH = 16
HCH = HALF // NCH
QCH = QUART // NCH
DEPTH = 6

CXB = 16
MSK_W0 = 26
MBC = 128
MBLK = N // MBC

_mesh = plsc.VectorSubcoreMesh(core_axis_name="c", subcore_axis_name="s")


@functools.partial(
    pl.kernel,
    mesh=_mesh,
    out_type=[
        jax.ShapeDtypeStruct((L, N, D), jnp.float32),
        jax.ShapeDtypeStruct((L, N), jnp.int32),
    ],
    scratch_types=[
        pltpu.VMEM_SHARED((NS, 3, NCH, D), jnp.float32),
        pltpu.VMEM((3, NCH, D), jnp.float32),
        pltpu.VMEM((CXB, D), jnp.float32),
        pltpu.VMEM((L, MBC + 64), jnp.int32),
        pltpu.VMEM((N_CTX, D), jnp.float32),
        pltpu.SemaphoreType.DMA,
        pltpu.SemaphoreType.DMA,
        pltpu.SemaphoreType.DMA,
        pltpu.SemaphoreType.DMA,
        pltpu.SemaphoreType.DMA,
        pltpu.SemaphoreType.DMA,
        pltpu.SemaphoreType.DMA,
        pltpu.SemaphoreType.DMA,
        pltpu.SemaphoreType.DMA,
        pltpu.SemaphoreType.DMA,
        pltpu.SemaphoreType.DMA,
        pltpu.SemaphoreType.DMA,
        pltpu.SemaphoreType.DMA,
        pltpu.SemaphoreType.DMA,
    ],
)
def _assemble(emb, ctx, msk, out_emb, out_msk, sp_v, v_v, b_v, m_v, c_v,
              semG0, semG1, semG2, semG3, semG4, semG5,
              semS0, semS1, semS2, semS3, semS4, semS5, semC, semM):
    sid = lax.axis_index("s")
    wid = sid * NC + lax.axis_index("c")
    semG = (semG0, semG1, semG2, semG3, semG4, semG5)
    semS = (semS0, semS1, semS2, semS3, semS4, semS5)

    def slot_buf(s):
        return sp_v.at[sid, s] if s < 3 else v_v.at[s - 3]

    def gchunk(s, l_src, n0):
        return pltpu.make_async_copy(
            emb.at[l_src, pl.ds(n0, NCH)], slot_buf(s), semG[s])

    def schunk(s, l_dst, n0):
        return pltpu.make_async_copy(
            slot_buf(s), out_emb.at[l_dst, pl.ds(n0, NCH)], semS[s])

    def run_unit(l_src, l_dst, nbase, nchunks):
        for c in range(nchunks):
            s = c % DEPTH
            if c >= DEPTH:
                schunk(s, l_dst, nbase + NCH * (c - DEPTH)).wait()
            gchunk(s, l_src, nbase + NCH * c).start()
            if c >= 2:
                s2 = (c - 2) % DEPTH
                gchunk(s2, l_src, nbase + NCH * (c - 2)).wait()
                schunk(s2, l_dst, nbase + NCH * (c - 2)).start()
        for c in (nchunks - 2, nchunks - 1):
            s2 = c % DEPTH
            gchunk(s2, l_src, nbase + NCH * c).wait()
            schunk(s2, l_dst, nbase + NCH * c).start()
        for c in range(nchunks - DEPTH, nchunks):
            schunk(c % DEPTH, l_dst, nbase + NCH * c).wait()

    def unit_body(k, carry):
        u = wid + NW * k
        slab = u // 2
        l_dst = jnp.where(slab == 0, 0, slab + N_CTX)
        run_unit(slab, l_dst, (u % 2) * HALF, HCH)
        return carry

    lax.fori_loop(0, 4, unit_body, 0)

    @pl.when(wid < 20)
    def _extras():
        e = 128 + wid // 2
        slab = e // 2
        nbase = (e % 2) * HALF + (wid % 2) * QUART
        run_unit(slab, slab + N_CTX, nbase, QCH)

    j = wid // 4
    l_ctx = CTX_POS + j
    nb_ctx = (wid % 4) * QUART
    pltpu.sync_copy(ctx, c_v)

    def repl(kk, carry):
        val = c_v[j, pl.ds(kk * LANES, LANES)]
        for r in range(CXB):
            b_v[r, pl.ds(kk * LANES, LANES)] = val
        return carry

    lax.fori_loop(0, CHD, repl, 0)
    ctx_cps = [
        pltpu.make_async_copy(
            b_v, out_emb.at[l_ctx, pl.ds(nb_ctx + CXB * c, CXB)], semC)
        for c in range(QUART // CXB)
    ]
    for cp in ctx_cps:
        cp.start()
    for cp in ctx_cps:
        cp.wait()

    ones16 = jnp.full((LANES,), 1, jnp.int32)

    def mask_block(n0, col0, ncols):
        gin = pltpu.make_async_copy(
            msk.at[pl.ds(0, L), pl.ds(n0, ncols)],
            m_v.at[pl.ds(0, L), pl.ds(col0, ncols)], semM)
        gin.start()
        gin.wait()
        nck = ncols // LANES

        def shrow(t, carry):
            i = (L - 1) - t
            for kk in range(nck):
                m_v[i, pl.ds(col0 + kk * LANES, LANES)] = \
                    m_v[i - N_CTX, pl.ds(col0 + kk * LANES, LANES)]
            return carry

        lax.fori_loop(0, L - CTX_POS - N_CTX, shrow, 0)
        for r in range(CTX_POS, CTX_POS + N_CTX):
            for kk in range(nck):
                m_v[r, pl.ds(col0 + kk * LANES, LANES)] = ones16
        gout = pltpu.make_async_copy(
            m_v.at[pl.ds(0, L), pl.ds(col0, ncols)],
            out_msk.at[pl.ds(0, L), pl.ds(n0, ncols)], semM)
        gout.start()
        gout.wait()

    @pl.when(jnp.logical_and(wid >= MSK_W0, wid < NW))
    def _mask():
        def mb(k, carry):
            u = (wid - MSK_W0) + 6 * k
            mask_block(MBC * u, 0, MBC)
            return carry

        lax.fori_loop(0, 2, mb, 0)

    @pl.when(wid == MSK_W0)
    def _mask_last():
        mask_block(MBC * MBLK, MBC, N - MBC * MBLK)


def kernel(token_emb_fixed, ctx, attn_mask, positional_embedding):
    del positional_embedding
    emb_t = jnp.transpose(token_emb_fixed, (1, 0, 2))
    msk_t = attn_mask.T
    out_t, outm_t = _assemble(emb_t, ctx, msk_t)
    return jnp.transpose(out_t, (1, 0, 2)), outm_t.T


# speedup vs baseline: 1.0408x; 1.0408x over previous
import functools

import jax
import jax.numpy as jnp
from jax import lax
from jax.experimental import pallas as pl
from jax.experimental.pallas import tpu as pltpu
from jax.experimental.pallas import tpu_sc as plsc

N, L, D = 1600, 77, 768
N_CTX = 8
CTX_POS = 1
NC, NS = 2, 16
NW = NC * NS
LANES = 16
CHD = D // LANES

SLABS = L - N_CTX
HALF = N // 2
QUART = N // 4
NCH = 40
HCH = HALF // NCH
QCH = QUART // NCH
DEPTH = 3

CXB = 16
MSK_W0 = 26
MBC = 128
MBLK = N // MBC

_mesh = plsc.VectorSubcoreMesh(core_axis_name="c", subcore_axis_name="s")


@functools.partial(
    pl.kernel,
    mesh=_mesh,
    out_type=[
        jax.ShapeDtypeStruct((L, N, D), jnp.float32),
        jax.ShapeDtypeStruct((L, N), jnp.int32),
    ],
    scratch_types=[
        pltpu.VMEM_SHARED((NS, DEPTH, NCH, D), jnp.float32),
        pltpu.VMEM((CXB, D), jnp.float32),
        pltpu.VMEM((L, MBC + 64), jnp.int32),
        pltpu.VMEM((N_CTX, D), jnp.float32),
        pltpu.SemaphoreType.DMA,
        pltpu.SemaphoreType.DMA,
        pltpu.SemaphoreType.DMA,
        pltpu.SemaphoreType.DMA,
        pltpu.SemaphoreType.DMA,
        pltpu.SemaphoreType.DMA,
        pltpu.SemaphoreType.DMA,
        pltpu.SemaphoreType.DMA,
    ],
)
def _assemble(emb, ctx, msk, out_emb, out_msk, sp_v, b_v, m_v, c_v,
              semG0, semG1, semG2, semS0, semS1, semS2, semC, semM):
    sid = lax.axis_index("s")
    wid = sid * NC + lax.axis_index("c")
    semG = (semG0, semG1, semG2)
    semS = (semS0, semS1, semS2)

    def gchunk(s, l_src, n0):
        return pltpu.make_async_copy(
            emb.at[l_src, pl.ds(n0, NCH)], sp_v.at[sid, s], semG[s])

    def schunk(s, l_dst, n0):
        return pltpu.make_async_copy(
            sp_v.at[sid, s], out_emb.at[l_dst, pl.ds(n0, NCH)], semS[s])

    def run_unit(l_src, l_dst, nbase, nchunks):
        for c in range(nchunks):
            s = c % DEPTH
            if c >= DEPTH:
                schunk(s, l_dst, nbase + NCH * (c - DEPTH)).wait()
            gchunk(s, l_src, nbase + NCH * c).start()
            if c >= 2:
                s2 = (c - 2) % DEPTH
                gchunk(s2, l_src, nbase + NCH * (c - 2)).wait()
                schunk(s2, l_dst, nbase + NCH * (c - 2)).start()
        for c in (nchunks - 2, nchunks - 1):
            s2 = c % DEPTH
            gchunk(s2, l_src, nbase + NCH * c).wait()
            schunk(s2, l_dst, nbase + NCH * c).start()
        for c in range(nchunks - DEPTH, nchunks):
            schunk(c % DEPTH, l_dst, nbase + NCH * c).wait()

    def unit_body(k, carry):
        u = wid + NW * k
        slab = u // 2
        l_dst = jnp.where(slab == 0, 0, slab + N_CTX)
        run_unit(slab, l_dst, (u % 2) * HALF, HCH)
        return carry

    lax.fori_loop(0, 4, unit_body, 0)

    @pl.when(wid < 20)
    def _extras():
        e = 128 + wid // 2
        slab = e // 2
        nbase = (e % 2) * HALF + (wid % 2) * QUART
        run_unit(slab, slab + N_CTX, nbase, QCH)

    j = wid // 4
    l_ctx = CTX_POS + j
    nb_ctx = (wid % 4) * QUART
    pltpu.sync_copy(ctx, c_v)

    def repl(kk, carry):
        val = c_v[j, pl.ds(kk * LANES, LANES)]
        for r in range(CXB):
            b_v[r, pl.ds(kk * LANES, LANES)] = val
        return carry

    lax.fori_loop(0, CHD, repl, 0)
    ctx_cps = [
        pltpu.make_async_copy(
            b_v, out_emb.at[l_ctx, pl.ds(nb_ctx + CXB * c, CXB)], semC)
        for c in range(QUART // CXB)
    ]
    for cp in ctx_cps:
        cp.start()
    for cp in ctx_cps:
        cp.wait()

    ones16 = jnp.full((LANES,), 1, jnp.int32)

    def mask_block(n0, col0, ncols):
        gin = pltpu.make_async_copy(
            msk.at[pl.ds(0, L), pl.ds(n0, ncols)],
            m_v.at[pl.ds(0, L), pl.ds(col0, ncols)], semM)
        gin.start()
        gin.wait()
        nck = ncols // LANES

        def shrow(t, carry):
            i = (L - 1) - t
            for kk in range(nck):
                m_v[i, pl.ds(col0 + kk * LANES, LANES)] = \
                    m_v[i - N_CTX, pl.ds(col0 + kk * LANES, LANES)]
            return carry

        lax.fori_loop(0, L - CTX_POS - N_CTX, shrow, 0)
        for r in range(CTX_POS, CTX_POS + N_CTX):
            for kk in range(nck):
                m_v[r, pl.ds(col0 + kk * LANES, LANES)] = ones16
        gout = pltpu.make_async_copy(
            m_v.at[pl.ds(0, L), pl.ds(col0, ncols)],
            out_msk.at[pl.ds(0, L), pl.ds(n0, ncols)], semM)
        gout.start()
        gout.wait()

    @pl.when(jnp.logical_and(wid >= MSK_W0, wid < NW))
    def _mask():
        def mb(k, carry):
            u = (wid - MSK_W0) + 6 * k
            mask_block(MBC * u, 0, MBC)
            return carry

        lax.fori_loop(0, 2, mb, 0)

    @pl.when(wid == MSK_W0)
    def _mask_last():
        mask_block(MBC * MBLK, MBC, N - MBC * MBLK)


def kernel(token_emb_fixed, ctx, attn_mask, positional_embedding):
    del positional_embedding
    emb_t = jnp.transpose(token_emb_fixed, (1, 0, 2))
    msk_t = attn_mask.T
    out_t, outm_t = _assemble(emb_t, ctx, msk_t)
    return jnp.transpose(out_t, (1, 0, 2)), outm_t.T
